# Initial kernel scaffold; baseline (speedup 1.0000x reference)
import functools

import numpy as np
import jax
import jax.numpy as jnp
from jax import lax
from jax.experimental import pallas as pl
from jax.experimental.pallas import tpu as pltpu
from jax.experimental.pallas import tpu_sc as plsc

NCH = 16
ACT_NORM = 1.6791767


def _mlp_body(x_ref, w1_ref, w2_ref, w3_ref, w4_ref, out_ref):
    h = x_ref[...]
    h = jax.nn.silu(jnp.dot(h, w1_ref[...], preferred_element_type=jnp.float32)) * ACT_NORM
    h = jax.nn.silu(jnp.dot(h, w2_ref[...], preferred_element_type=jnp.float32)) * ACT_NORM
    h = jax.nn.silu(jnp.dot(h, w3_ref[...], preferred_element_type=jnp.float32)) * ACT_NORM
    out_ref[...] = jnp.dot(h, w4_ref[...], preferred_element_type=jnp.float32)


def _run_mlp(lenght, W1, W2, W3, W4, block):
    E = lenght.shape[0]
    grid = (E // block,)
    return pl.pallas_call(
        _mlp_body,
        grid=grid,
        in_specs=[
            pl.BlockSpec((block, 8), lambda i: (i, 0)),
            pl.BlockSpec((8, 6), lambda i: (0, 0)),
            pl.BlockSpec((6, 6), lambda i: (0, 0)),
            pl.BlockSpec((6, 6), lambda i: (0, 0)),
            pl.BlockSpec((6, 48), lambda i: (0, 0)),
        ],
        out_specs=pl.BlockSpec((block, 48), lambda i: (i, 0)),
        out_shape=jax.ShapeDtypeStruct((E, 48), jnp.float32),
    )(lenght, W1, W2, W3, W4)


def _lin_body(m_ref, wb_ref, o_ref):
    o_ref[...] = jnp.dot(m_ref[...], wb_ref[...], preferred_element_type=jnp.float32)


def _run_linear(msg, Wb, block):
    N = msg.shape[0]
    grid = (N // block,)
    return pl.pallas_call(
        _lin_body,
        grid=grid,
        in_specs=[
            pl.BlockSpec((block, 144), lambda i: (i, 0)),
            pl.BlockSpec((144, 144), lambda i: (0, 0)),
        ],
        out_specs=pl.BlockSpec((block, 144), lambda i: (i, 0)),
        out_shape=jax.ShapeDtypeStruct((N, 144), jnp.float32),
    )(msg, Wb)


_NTILES = 16
_NCORES = 2
_B = 400
_GCH = 80


@functools.lru_cache(maxsize=None)
def _build_sc(E, NQ):
    B = _B
    NG = B // _GCH
    EPT = E // _NTILES
    NBLK = EPT // B
    ACC = NQ + _NTILES
    ROWS_OUT = NQ // _NTILES
    ROWS_ZERO = ACC // _NTILES

    mesh = plsc.VectorSubcoreMesh(core_axis_name="c", subcore_axis_name="s")

    @functools.partial(
        pl.kernel,
        out_type=jax.ShapeDtypeStruct((4 * NQ, 144), jnp.float32),
        mesh=mesh,
        scratch_types=[
            pltpu.VMEM((B, 48), jnp.float32),
            pltpu.VMEM((B, 9), jnp.float32),
            pltpu.VMEM((B, 16), jnp.float32),
            pltpu.VMEM((B, 144), jnp.float32),
            pltpu.VMEM((B,), jnp.int32),
            pltpu.VMEM((B,), jnp.int32),
            pltpu.VMEM((B // _GCH, _GCH), jnp.int32),
            pltpu.VMEM_SHARED((NQ + _NTILES, 144), jnp.float32),
            pltpu.SemaphoreType.DMA,
        ],
    )
    def sc_kernel(w_hbm, ea_hbm, snd_hbm, rcv_hbm, nf_hbm, out_hbm,
                  u_v, ea_v, xs_v, mij_v, rcv_v, snd_v, lidx_v, acc, sem):
        c = lax.axis_index("c")
        s = lax.axis_index("s")
        zeros16 = jnp.zeros((16,), jnp.float32)

        def run_pass(p):
            q = 2 * p + c
            base_node = q * NQ

            def zrow(e, carry):
                for k in range(9):
                    mij_v[e, pl.ds(k * 16, 16)] = zeros16
                return carry
            lax.fori_loop(0, min(B, ROWS_ZERO), zrow, 0)
            done = 0
            while done < ROWS_ZERO:
                n = min(B, ROWS_ZERO - done)
                pltpu.sync_copy(mij_v.at[pl.ds(0, n)],
                                acc.at[pl.ds(s * ROWS_ZERO + done, n)])
                done += n
            plsc.subcore_barrier()

            def blk(b, carry):
                e0 = s * EPT + b * B
                pltpu.sync_copy(rcv_hbm.at[pl.ds(e0, B)], rcv_v)
                pltpu.sync_copy(snd_hbm.at[pl.ds(e0, B)], snd_v)
                pltpu.sync_copy(w_hbm.at[pl.ds(e0, B)], u_v)
                pltpu.sync_copy(ea_hbm.at[pl.ds(e0, B)], ea_v)

                cps = [
                    pltpu.async_copy(
                        nf_hbm.at[snd_v.at[pl.ds(j * _GCH, _GCH)]],
                        xs_v.at[pl.ds(j * _GCH, _GCH)],
                        sem,
                    )
                    for j in range(NG)
                ]
                for cp in cps:
                    cp.wait()

                def lid(g, carry2):
                    r = rcv_v[pl.ds(g * 16, 16)]
                    loc = r - base_node
                    m = (loc >= 0) & (loc < NQ)
                    lidx_v[g // (_GCH // 16), pl.ds((g % (_GCH // 16)) * 16, 16)] = (
                        jnp.where(m, loc, NQ))
                    return carry2
                lax.fori_loop(0, B // 16, lid, 0)

                def edge(e, carry2):
                    xsr = xs_v[e, :]
                    xw0 = xsr * u_v[e, pl.ds(0, 16)]
                    xw1 = xsr * u_v[e, pl.ds(16, 16)]
                    xw2 = xsr * u_v[e, pl.ds(32, 16)]
                    xws = (xw0, xw1, xw1, xw1, xw2, xw2, xw2, xw2, xw2)
                    for k in range(9):
                        mij_v[e, pl.ds(k * 16, 16)] = xws[k] * ea_v[e, k]
                    return carry2
                lax.fori_loop(0, B, edge, 0)

                for j in range(NG):
                    pltpu.sync_copy(mij_v.at[pl.ds(j * _GCH, _GCH)],
                                    acc.at[lidx_v.at[j]], add=True)
                return carry
            lax.fori_loop(0, NBLK, blk, 0)
            plsc.subcore_barrier()

            pltpu.sync_copy(acc.at[pl.ds(s * ROWS_OUT, ROWS_OUT)],
                            out_hbm.at[pl.ds(q * NQ + s * ROWS_OUT, ROWS_OUT)])
            plsc.subcore_barrier()

        run_pass(0)
        run_pass(1)

    return sc_kernel


def kernel(lenght, node_features, edge_attributes, edge_index,
           W_fc1, W_fc2, W_fc3, W_fc4, W_l0, W_l1, W_l2):
    E = lenght.shape[0]
    N = node_features.shape[0]

    NQ = ((N + 4 * _NTILES - 1) // (4 * _NTILES)) * _NTILES

    W1 = W_fc1 / np.sqrt(W_fc1.shape[0])
    W2 = W_fc2 / np.sqrt(W_fc2.shape[0])
    W3 = W_fc3 / np.sqrt(W_fc3.shape[0])
    W4 = W_fc4 / np.sqrt(W_fc4.shape[0])
    w = _run_mlp(lenght, W1, W2, W3, W4, block=4000)

    snd = edge_index[0]
    rcv = edge_index[1]
    msg_pad = _build_sc(E, NQ)(w, edge_attributes, snd, rcv, node_features)
    msg = msg_pad[:N]

    inv = 1.0 / np.sqrt(NCH)
    Wb = jnp.zeros((144, 144), jnp.float32)
    Wb = Wb.at[0:16, 0:16].set(W_l0 * inv)
    for ci in range(3):
        Wb = Wb.at[16 * (1 + ci):16 * (2 + ci), 16 + ci:64:3].set(W_l1 * inv)
    for ci in range(5):
        Wb = Wb.at[16 * (4 + ci):16 * (5 + ci), 64 + ci:144:5].set(W_l2 * inv)
    return _run_linear(msg, Wb, block=1000)


# speedup vs baseline: 1.3127x; 1.3127x over previous
import functools

import numpy as np
import jax
import jax.numpy as jnp
from jax import lax
from jax.experimental import pallas as pl
from jax.experimental.pallas import tpu as pltpu
from jax.experimental.pallas import tpu_sc as plsc

NCH = 16
ACT_NORM = 1.6791767


def _mlp_body(x_ref, ea_ref, w1_ref, w2_ref, w3_ref, w4_ref, out_ref):
    h = x_ref[...]
    h = jax.nn.silu(jnp.dot(h, w1_ref[...], preferred_element_type=jnp.float32)) * ACT_NORM
    h = jax.nn.silu(jnp.dot(h, w2_ref[...], preferred_element_type=jnp.float32)) * ACT_NORM
    h = jax.nn.silu(jnp.dot(h, w3_ref[...], preferred_element_type=jnp.float32)) * ACT_NORM
    w = jnp.dot(h, w4_ref[...], preferred_element_type=jnp.float32)
    pad = jnp.zeros((w.shape[0], 7), jnp.float32)
    out_ref[...] = jnp.concatenate([w, ea_ref[...], pad], axis=1)


def _run_mlp(lenght, edge_attributes, W1, W2, W3, W4, block):
    E = lenght.shape[0]
    grid = (E // block,)
    return pl.pallas_call(
        _mlp_body,
        grid=grid,
        in_specs=[
            pl.BlockSpec((block, 8), lambda i: (i, 0)),
            pl.BlockSpec((block, 9), lambda i: (i, 0)),
            pl.BlockSpec((8, 6), lambda i: (0, 0)),
            pl.BlockSpec((6, 6), lambda i: (0, 0)),
            pl.BlockSpec((6, 6), lambda i: (0, 0)),
            pl.BlockSpec((6, 48), lambda i: (0, 0)),
        ],
        out_specs=pl.BlockSpec((block, 64), lambda i: (i, 0)),
        out_shape=jax.ShapeDtypeStruct((E, 64), jnp.float32),
    )(lenght, edge_attributes, W1, W2, W3, W4)


def _lin_body(m_ref, wb_ref, o_ref):
    o_ref[...] = jnp.dot(m_ref[...], wb_ref[...], preferred_element_type=jnp.float32)


def _run_linear(msg, Wb, block):
    N = msg.shape[0]
    grid = (N // block,)
    return pl.pallas_call(
        _lin_body,
        grid=grid,
        in_specs=[
            pl.BlockSpec((block, 144), lambda i: (i, 0)),
            pl.BlockSpec((144, 144), lambda i: (0, 0)),
        ],
        out_specs=pl.BlockSpec((block, 144), lambda i: (i, 0)),
        out_shape=jax.ShapeDtypeStruct((N, 144), jnp.float32),
    )(msg, Wb)


_NTILES = 16
_NCORES = 2
_B = 80
_SUB = 16


@functools.lru_cache(maxsize=None)
def _build_sc(E, NQ):
    B = _B
    NG = B // _SUB
    EPT = E // _NTILES
    NBLK = EPT // B
    ACC = NQ + 16
    ROWS_OUT = NQ // _NTILES

    mesh = plsc.VectorSubcoreMesh(core_axis_name="c", subcore_axis_name="s")

    @functools.partial(
        pl.kernel,
        out_type=jax.ShapeDtypeStruct((4 * NQ, 144), jnp.float32),
        mesh=mesh,
        scratch_types=[
            pltpu.VMEM((B, 64), jnp.float32),
            pltpu.VMEM((B, 16), jnp.float32),
            pltpu.VMEM((_SUB, 144), jnp.float32),
            pltpu.VMEM((B,), jnp.int32),
            pltpu.VMEM((B,), jnp.int32),
            pltpu.VMEM((NG, _SUB), jnp.int32),
            pltpu.VMEM_SHARED((NQ + 16, 144), jnp.float32),
            pltpu.SemaphoreType.DMA,
        ],
        compiler_params=pltpu.CompilerParams(use_tc_tiling_on_sc=False),
    )
    def sc_kernel(u_hbm, snd_hbm, rcv_hbm, nf_hbm, out_hbm,
                  u_v, xs_v, mij_v, rcv_v, snd_v, lidx_v, acc, sem):
        c = lax.axis_index("c")
        s = lax.axis_index("s")
        zeros16 = jnp.zeros((16,), jnp.float32)

        def run_pass(p):
            q = 2 * p + c
            base_node = q * NQ

            def zrow(e, carry):
                for k in range(9):
                    mij_v[e, pl.ds(k * 16, 16)] = zeros16
                return carry
            lax.fori_loop(0, _SUB, zrow, 0)
            done = 0
            while done < ROWS_OUT:
                n = min(_SUB, ROWS_OUT - done)
                pltpu.sync_copy(mij_v.at[pl.ds(0, n)],
                                acc.at[pl.ds(s * ROWS_OUT + done, n)])
                done += n
            plsc.subcore_barrier()

            def blk(b, carry):
                e0 = s * EPT + b * B
                pltpu.sync_copy(rcv_hbm.at[pl.ds(e0, B)], rcv_v)
                pltpu.sync_copy(snd_hbm.at[pl.ds(e0, B)], snd_v)
                pltpu.sync_copy(u_hbm.at[pl.ds(e0, B)], u_v)

                pltpu.async_copy(nf_hbm.at[snd_v], xs_v, sem).wait()

                def lid(g, carry2):
                    r = rcv_v[pl.ds(g * 16, 16)]
                    loc = r - base_node
                    m = (loc >= 0) & (loc < NQ)
                    lidx_v[g, :] = jnp.where(m, loc, NQ)
                    return carry2
                lax.fori_loop(0, NG, lid, 0)

                for g in range(NG):
                    def edge(j, carry2, g=g):
                        e = g * _SUB + j
                        xsr = xs_v[e, :]
                        xw0 = xsr * u_v[e, pl.ds(0, 16)]
                        xw1 = xsr * u_v[e, pl.ds(16, 16)]
                        xw2 = xsr * u_v[e, pl.ds(32, 16)]
                        sh = u_v[e, pl.ds(48, 16)]
                        xws = (xw0, xw1, xw1, xw1, xw2, xw2, xw2, xw2, xw2)
                        for k in range(9):
                            mij_v[j, pl.ds(k * 16, 16)] = xws[k] * sh[k]
                        return carry2
                    lax.fori_loop(0, _SUB, edge, 0)
                    pltpu.sync_copy(mij_v, acc.at[lidx_v.at[g]], add=True)
                return carry
            lax.fori_loop(0, NBLK, blk, 0)
            plsc.subcore_barrier()

            pltpu.sync_copy(acc.at[pl.ds(s * ROWS_OUT, ROWS_OUT)],
                            out_hbm.at[pl.ds(q * NQ + s * ROWS_OUT, ROWS_OUT)])
            plsc.subcore_barrier()

        run_pass(0)
        run_pass(1)

    return sc_kernel


def kernel(lenght, node_features, edge_attributes, edge_index,
           W_fc1, W_fc2, W_fc3, W_fc4, W_l0, W_l1, W_l2):
    E = lenght.shape[0]
    N = node_features.shape[0]

    NQ = ((N + 4 * 128 - 1) // (4 * 128)) * 128

    W1 = W_fc1 / np.sqrt(W_fc1.shape[0])
    W2 = W_fc2 / np.sqrt(W_fc2.shape[0])
    W3 = W_fc3 / np.sqrt(W_fc3.shape[0])
    W4 = W_fc4 / np.sqrt(W_fc4.shape[0])
    u = _run_mlp(lenght, edge_attributes, W1, W2, W3, W4, block=4000)

    snd = edge_index[0]
    rcv = edge_index[1]
    msg_pad = _build_sc(E, NQ)(u, snd, rcv, node_features)
    msg = msg_pad[:N]

    inv = 1.0 / np.sqrt(NCH)
    Wb = jnp.zeros((144, 144), jnp.float32)
    Wb = Wb.at[0:16, 0:16].set(W_l0 * inv)
    for ci in range(3):
        Wb = Wb.at[16 * (1 + ci):16 * (2 + ci), 16 + ci:64:3].set(W_l1 * inv)
    for ci in range(5):
        Wb = Wb.at[16 * (4 + ci):16 * (5 + ci), 64 + ci:144:5].set(W_l2 * inv)
    return _run_linear(msg, Wb, block=1000)
